# Initial kernel scaffold; baseline (speedup 1.0000x reference)
import jax
import jax.numpy as jnp
from jax.experimental import pallas as pl


def kernel(x, force_drop, noise):
    raise NotImplementedError("write your pallas kernel here")


# speedup vs baseline: 3.2777x; 3.2777x over previous
import functools

import jax
import jax.numpy as jnp
from jax import lax
from jax.experimental import pallas as pl
from jax.experimental.pallas import tpu as pltpu
from jax.experimental.pallas import tpu_sc as plsc

BATCH = 64
SEQ = 1025
PATCH = 1024
DIM = 768
KEEP = 512
OUT = KEEP + 1

NC, NS, L = 2, 16, 16
NW = NC * NS
RPT = BATCH // NW
TILE_ROWS = RPT * OUT
C = 57
NCHUNK = TILE_ROWS // C
assert NCHUNK * C == TILE_ROWS
NCHV = PATCH // L


def _body(x_hbm, noise_hbm, out_hbm, noise_v, idx_v, buf0, buf1, sem0, sem1):
    wid = lax.axis_index("s") * NC + lax.axis_index("c")

    zeros = jnp.zeros((L,), jnp.int32)
    ones = jnp.full((L,), 1, jnp.int32)
    kvec = jnp.full((L,), KEEP, jnp.int32)
    lanes = lax.iota(jnp.int32, L)

    for r in range(RPT):
        b = wid * RPT + r
        pltpu.sync_copy(noise_hbm.at[b], noise_v)

        def count_le(t_vec):
            def cbody(i, acc):
                bits = plsc.bitcast(noise_v[pl.ds(i * L, L)], jnp.int32)
                return acc + plsc.all_reduce_population_count(bits <= t_vec)

            return lax.fori_loop(0, NCHV, cbody, zeros, unroll=4)

        def sbody(_, lohi):
            lo, hi = lohi
            mid = lo + lax.shift_right_logical(hi - lo, 1)
            pred = count_le(mid) >= kvec
            return jnp.where(pred, lo, mid + 1), jnp.where(pred, mid, hi)

        _, tstar = lax.fori_loop(
            0, 31, sbody, (zeros, jnp.full((L,), 0x7FFFFFFF, jnp.int32))
        )

        def cbody_lt(i, acc):
            bits = plsc.bitcast(noise_v[pl.ds(i * L, L)], jnp.int32)
            return acc + plsc.all_reduce_population_count(bits < tstar)

        m = lax.fori_loop(0, NCHV, cbody_lt, zeros, unroll=4)
        need_eq = kvec - m

        p0 = jnp.full((L,), r * OUT, jnp.int32)
        plsc.store_scatter(
            idx_v,
            [p0 // C, p0 % C],
            jnp.full((L,), b * SEQ, jnp.int32),
            mask=lanes == zeros,
        )

        def compact(i, carry):
            kept, eqs = carry
            bits = plsc.bitcast(noise_v[pl.ds(i * L, L)], jnp.int32)
            is_lt = bits < tstar
            is_eq = bits == tstar
            eq_i = jnp.where(is_eq, ones, zeros)
            eq_rank = plsc.cumsum(eq_i) - eq_i + eqs
            keep = is_lt | (is_eq & (eq_rank < need_eq))
            k_i = jnp.where(keep, ones, zeros)
            pos = plsc.cumsum(k_i) - k_i + kept
            p = pos + (r * OUT + 1)
            gidx = (i * L + b * SEQ) + lanes
            plsc.store_scatter(idx_v, [p // C, p % C], gidx, mask=keep)
            return (
                kept + plsc.all_reduce_population_count(keep),
                eqs + plsc.all_reduce_population_count(is_eq),
            )

        lax.fori_loop(0, NCHV, compact, (zeros, zeros))

    out_base = wid * TILE_ROWS

    def issue(c, buf, sem):
        pltpu.async_copy(x_hbm.at[idx_v.at[c]], buf, sem)

    def drain(c, buf, sem):
        pltpu.make_async_copy(x_hbm.at[idx_v.at[c]], buf, sem).wait()

    issue(0, buf0, sem0)

    def gbody(cc, _):
        c = cc * 2
        issue(c + 1, buf1, sem1)
        drain(c, buf0, sem0)
        pltpu.sync_copy(buf0, out_hbm.at[pl.ds(out_base + c * C, C)])

        @pl.when(cc + 1 < NCHUNK // 2)
        def _():
            issue(c + 2, buf0, sem0)

        drain(c + 1, buf1, sem1)
        pltpu.sync_copy(buf1, out_hbm.at[pl.ds(out_base + (c + 1) * C, C)])
        return 0

    lax.fori_loop(0, NCHUNK // 2, gbody, 0)


@jax.jit
def _run(x_flat, noise):
    mesh = plsc.VectorSubcoreMesh(
        core_axis_name="c", subcore_axis_name="s", num_cores=NC, num_subcores=NS
    )
    f = pl.kernel(
        _body,
        out_type=jax.ShapeDtypeStruct((BATCH * OUT, DIM), jnp.float32),
        mesh=mesh,
        scratch_types=[
            pltpu.VMEM((PATCH,), jnp.float32),
            pltpu.VMEM((NCHUNK, C), jnp.int32),
            pltpu.VMEM((C, DIM), jnp.float32),
            pltpu.VMEM((C, DIM), jnp.float32),
            pltpu.SemaphoreType.DMA,
            pltpu.SemaphoreType.DMA,
        ],
        compiler_params=pltpu.CompilerParams(
            use_tc_tiling_on_sc=False, needs_layout_passes=False
        ),
    )
    return f(x_flat, noise)


def kernel(x, force_drop, noise):
    del force_drop
    out = _run(x.reshape(BATCH * SEQ, DIM), noise)
    return out.reshape(BATCH, OUT, DIM)
